# baseline (speedup 1.0000x reference)
import functools
import jax
import jax.numpy as jnp
from jax import lax
from jax.experimental import pallas as pl
from jax.experimental.pallas import tpu as pltpu
from jax.experimental.pallas import tpu_sc as plsc

BATCH = 4
SEQ = 8192
D_MODEL = 1024
NUM_CORES = 2
NUM_SUBCORES = 16
NUM_WORKERS = NUM_CORES * NUM_SUBCORES
ROWS_PER_WORKER = SEQ // NUM_WORKERS
TILE_ROWS = 16
TILES_PER_WORKER = ROWS_PER_WORKER // TILE_ROWS
LANES = 16
VECS_PER_ROW = D_MODEL // LANES


def _sc_body(x_hbm, emb_hbm, out_hbm, emb_v, x_v, sem):
    wid = lax.axis_index("s") * NUM_CORES + lax.axis_index("c")
    base = wid * ROWS_PER_WORKER

    def tile_body(t, carry):
        row0 = base + t * TILE_ROWS
        pltpu.async_copy(emb_hbm.at[pl.ds(row0, TILE_ROWS)], emb_v, sem).wait()

        def batch_body(b, carry2):
            pltpu.async_copy(x_hbm.at[b, pl.ds(row0, TILE_ROWS)], x_v, sem).wait()

            def row_body(r, carry3):
                def vec_body(i, carry4):
                    c0 = i * LANES
                    x_v[r, pl.ds(c0, LANES)] = (
                        x_v[r, pl.ds(c0, LANES)] + emb_v[r, pl.ds(c0, LANES)]
                    )
                    return carry4

                return lax.fori_loop(0, VECS_PER_ROW, vec_body, carry3)

            lax.fori_loop(0, TILE_ROWS, row_body, carry2)
            pltpu.async_copy(x_v, out_hbm.at[b, pl.ds(row0, TILE_ROWS)], sem).wait()
            return carry2

        lax.fori_loop(0, BATCH, batch_body, carry)
        return carry

    lax.fori_loop(0, TILES_PER_WORKER, tile_body, 0)


def kernel(x, emb_weight):
    mesh = plsc.VectorSubcoreMesh(core_axis_name="c", subcore_axis_name="s")
    run = functools.partial(
        pl.kernel,
        out_type=jax.ShapeDtypeStruct((BATCH, SEQ, D_MODEL), jnp.float32),
        mesh=mesh,
        scratch_types=[
            pltpu.VMEM((TILE_ROWS, D_MODEL), jnp.float32),
            pltpu.VMEM((TILE_ROWS, D_MODEL), jnp.float32),
            pltpu.SemaphoreType.DMA,
        ],
    )(_sc_body)
    return run(x, emb_weight)


# speedup vs baseline: 1.1125x; 1.1125x over previous
import functools
import jax
import jax.numpy as jnp
from jax import lax
from jax.experimental import pallas as pl
from jax.experimental.pallas import tpu as pltpu
from jax.experimental.pallas import tpu_sc as plsc

BATCH = 4
SEQ = 8192
D_MODEL = 1024
NUM_CORES = 2
NUM_SUBCORES = 16
NUM_WORKERS = NUM_CORES * NUM_SUBCORES
ROWS_PER_WORKER = SEQ // NUM_WORKERS
TILE_ROWS = 8
NTILES = ROWS_PER_WORKER // TILE_ROWS
LANES = 16
UNROLL = 8
COL_ITERS = D_MODEL // (LANES * UNROLL)


def _sc_body(x_hbm, emb_hbm, out_hbm,
             e_v0, e_v1, x_v0, x_v1,
             se0, se1, sx0, sx1, so0, so1):
    wid = lax.axis_index("s") * NUM_CORES + lax.axis_index("c")
    base = wid * ROWS_PER_WORKER

    groups = ((e_v0, x_v0, se0, sx0, so0), (e_v1, x_v1, se1, sx1, so1))

    def issue_in(t, g):
        e_v, x_v, se, sx, _ = groups[g]
        row0 = base + t * TILE_ROWS
        pltpu.async_copy(emb_hbm.at[pl.ds(row0, TILE_ROWS)], e_v, se)
        pltpu.async_copy(x_hbm.at[:, pl.ds(row0, TILE_ROWS), :], x_v, sx)

    def wait_in(g):
        e_v, x_v, se, sx, _ = groups[g]
        pltpu.make_async_copy(emb_hbm.at[pl.ds(base, TILE_ROWS)], e_v, se).wait()
        pltpu.make_async_copy(x_hbm.at[:, pl.ds(base, TILE_ROWS), :], x_v, sx).wait()

    def wait_out(g):
        _, x_v, _, _, so = groups[g]
        pltpu.make_async_copy(x_v, out_hbm.at[:, pl.ds(base, TILE_ROWS), :], so).wait()

    def compute_and_out(t, g):
        e_v, x_v, _, _, so = groups[g]
        row0 = base + t * TILE_ROWS

        def row_body(r, carry):
            def col_body(i, carry2):
                for b in range(BATCH):
                    for u in range(UNROLL):
                        c0 = (i * UNROLL + u) * LANES
                        x_v[b, r, pl.ds(c0, LANES)] = (
                            x_v[b, r, pl.ds(c0, LANES)] + e_v[r, pl.ds(c0, LANES)]
                        )
                return carry2

            return lax.fori_loop(0, COL_ITERS, col_body, carry)

        lax.fori_loop(0, TILE_ROWS, row_body, 0)
        pltpu.async_copy(x_v, out_hbm.at[:, pl.ds(row0, TILE_ROWS), :], so)

    issue_in(0, 0)
    issue_in(1, 1)
    wait_in(0)
    compute_and_out(0, 0)

    def loop_body(t2, carry):
        t_a = 1 + 2 * t2
        wait_out(0)
        issue_in(t_a + 1, 0)
        wait_in(1)
        compute_and_out(t_a, 1)
        wait_out(1)
        issue_in(t_a + 2, 1)
        wait_in(0)
        compute_and_out(t_a + 1, 0)
        return carry

    lax.fori_loop(0, (NTILES - 2) // 2, loop_body, 0)

    wait_in(1)
    compute_and_out(NTILES - 1, 1)
    wait_out(0)
    wait_out(1)


def kernel(x, emb_weight):
    mesh = plsc.VectorSubcoreMesh(core_axis_name="c", subcore_axis_name="s")
    run = functools.partial(
        pl.kernel,
        out_type=jax.ShapeDtypeStruct((BATCH, SEQ, D_MODEL), jnp.float32),
        mesh=mesh,
        scratch_types=[
            pltpu.VMEM((TILE_ROWS, D_MODEL), jnp.float32),
            pltpu.VMEM((TILE_ROWS, D_MODEL), jnp.float32),
            pltpu.VMEM((BATCH, TILE_ROWS, D_MODEL), jnp.float32),
            pltpu.VMEM((BATCH, TILE_ROWS, D_MODEL), jnp.float32),
            pltpu.SemaphoreType.DMA,
            pltpu.SemaphoreType.DMA,
            pltpu.SemaphoreType.DMA,
            pltpu.SemaphoreType.DMA,
            pltpu.SemaphoreType.DMA,
            pltpu.SemaphoreType.DMA,
        ],
    )(_sc_body)
    return run(x, emb_weight)


# speedup vs baseline: 3.8415x; 3.4531x over previous
import functools
import jax
import jax.numpy as jnp
from jax import lax
from jax.experimental import pallas as pl
from jax.experimental.pallas import tpu as pltpu
from jax.experimental.pallas import tpu_sc as plsc

BATCH = 4
SEQ = 8192
D_MODEL = 1024
NUM_CORES = 2
NUM_SUBCORES = 16
NUM_WORKERS = NUM_CORES * NUM_SUBCORES
ROWS_PER_WORKER = SEQ // NUM_WORKERS
TILE_ROWS = 4
NTILES = ROWS_PER_WORKER // TILE_ROWS
NGROUPS = 4
LANES = 16
VECS_PER_TILE = TILE_ROWS * D_MODEL // LANES
COLS = D_MODEL // LANES


def _sc_body(x_hbm, emb_hbm, out_hbm,
             e_v0, e_v1, e_v2, e_v3,
             x_v0, x_v1, x_v2, x_v3,
             se0, se1, se2, se3,
             sx0, sx1, sx2, sx3,
             so0, so1, so2, so3):
    wid = lax.axis_index("s") * NUM_CORES + lax.axis_index("c")
    base = wid * ROWS_PER_WORKER

    groups = (
        (e_v0, x_v0, se0, sx0, so0),
        (e_v1, x_v1, se1, sx1, so1),
        (e_v2, x_v2, se2, sx2, so2),
        (e_v3, x_v3, se3, sx3, so3),
    )

    def issue_in(t, g):
        e_v, x_v, se, sx, _ = groups[g]
        row0 = base + t * TILE_ROWS
        pltpu.async_copy(emb_hbm.at[pl.ds(row0, TILE_ROWS)], e_v, se)
        pltpu.async_copy(x_hbm.at[:, pl.ds(row0, TILE_ROWS), :], x_v, sx)

    def wait_in(g):
        e_v, x_v, se, sx, _ = groups[g]
        pltpu.make_async_copy(emb_hbm.at[pl.ds(base, TILE_ROWS)], e_v, se).wait()
        pltpu.make_async_copy(x_hbm.at[:, pl.ds(base, TILE_ROWS), :], x_v, sx).wait()

    def wait_out(g):
        _, x_v, _, _, so = groups[g]
        pltpu.make_async_copy(x_v, out_hbm.at[:, pl.ds(base, TILE_ROWS), :], so).wait()

    def compute_and_out(t, g):
        e_v, x_v, _, _, so = groups[g]
        row0 = base + t * TILE_ROWS

        def vec_body(i, carry):
            r = i // COLS
            c0 = (i % COLS) * LANES
            e = e_v[r, pl.ds(c0, LANES)]
            for b in range(BATCH):
                x_v[b, r, pl.ds(c0, LANES)] = x_v[b, r, pl.ds(c0, LANES)] + e
            return carry

        lax.fori_loop(0, VECS_PER_TILE, vec_body, 0)

        pltpu.async_copy(x_v, out_hbm.at[:, pl.ds(row0, TILE_ROWS), :], so)

    def stage(t, g):
        if t >= 2:
            wait_out((t - 2) % NGROUPS)
        if t + NGROUPS - 2 < NTILES:
            issue_in(t + NGROUPS - 2, (t - 2) % NGROUPS)
        wait_in(g)
        compute_and_out(t, g)

    issue_in(0, 0)
    issue_in(1, 1)
    stage(0, 0)
    stage(1, 1)

    def loop_body(k, carry):
        t0 = 2 + 4 * k
        for j in range(4):
            g = (2 + j) % NGROUPS
            t = t0 + j
            wait_out((g - 2) % NGROUPS)
            issue_in(t + NGROUPS - 2, (g - 2) % NGROUPS)
            wait_in(g)
            compute_and_out(t, g)
        return carry

    lax.fori_loop(0, (NTILES - 4) // 4, loop_body, 0)

    stage(NTILES - 2, (NTILES - 2) % NGROUPS)
    stage(NTILES - 1, (NTILES - 1) % NGROUPS)
    wait_out((NTILES - 2) % NGROUPS)
    wait_out((NTILES - 1) % NGROUPS)


def kernel(x, emb_weight):
    mesh = plsc.VectorSubcoreMesh(core_axis_name="c", subcore_axis_name="s")
    run = functools.partial(
        pl.kernel,
        out_type=jax.ShapeDtypeStruct((BATCH, SEQ, D_MODEL), jnp.float32),
        mesh=mesh,
        scratch_types=[
            pltpu.VMEM((TILE_ROWS, D_MODEL), jnp.float32),
            pltpu.VMEM((TILE_ROWS, D_MODEL), jnp.float32),
            pltpu.VMEM((TILE_ROWS, D_MODEL), jnp.float32),
            pltpu.VMEM((TILE_ROWS, D_MODEL), jnp.float32),
            pltpu.VMEM((BATCH, TILE_ROWS, D_MODEL), jnp.float32),
            pltpu.VMEM((BATCH, TILE_ROWS, D_MODEL), jnp.float32),
            pltpu.VMEM((BATCH, TILE_ROWS, D_MODEL), jnp.float32),
            pltpu.VMEM((BATCH, TILE_ROWS, D_MODEL), jnp.float32),
            pltpu.SemaphoreType.DMA,
            pltpu.SemaphoreType.DMA,
            pltpu.SemaphoreType.DMA,
            pltpu.SemaphoreType.DMA,
            pltpu.SemaphoreType.DMA,
            pltpu.SemaphoreType.DMA,
            pltpu.SemaphoreType.DMA,
            pltpu.SemaphoreType.DMA,
            pltpu.SemaphoreType.DMA,
            pltpu.SemaphoreType.DMA,
            pltpu.SemaphoreType.DMA,
            pltpu.SemaphoreType.DMA,
        ],
    )(_sc_body)
    return run(x, emb_weight)
